# Initial kernel scaffold; baseline (speedup 1.0000x reference)
import jax
import jax.numpy as jnp
from jax.experimental import pallas as pl


def kernel(x, node_roi, edge_index, edge_attr, batch, x2, roi2, edge_index2, edge_attr2, batch2, params):
    raise NotImplementedError("write your pallas kernel here")


# speedup vs baseline: 7.6467x; 7.6467x over previous
import functools

import jax
import jax.numpy as jnp
from jax import lax
from jax.experimental import pallas as pl
from jax.experimental.pallas import tpu as pltpu
from jax.experimental.pallas import tpu_sc as plsc

N1 = 10000; E1 = 320000; N2 = 1184; E2 = 175232
D = 128; H = 4; NR = 148; B = 8; OUT = 2; HID = 1000
NSEG = B * NR

NC, NS = 2, 16
NW = NC * NS
CK = 128

@functools.lru_cache(maxsize=None)
def _get_mesh():
    return plsc.VectorSubcoreMesh(core_axis_name="c", subcore_axis_name="s",
                                  num_cores=NC, num_subcores=NS)


def _worker_id():
    return lax.axis_index("c") * NS + lax.axis_index("s")


def _zero_vmem_rows(ref, nrows):
    zeros = jnp.zeros((16,), jnp.float32)

    def body(r, _):
        for g in range(8):
            ref[r, pl.ds(16 * g, 16)] = zeros
        return _

    lax.fori_loop(0, nrows, body, None)


def _zero_shared_slice(shared, zbuf, zrows, row0, nrows):
    off = 0
    while off < nrows:
        sz = min(zrows, nrows - off)
        pltpu.sync_copy(zbuf.at[pl.ds(0, sz)],
                        shared.at[pl.ds(row0 + off, sz)])
        off += sz


TOTBINS = N1 + N2 + NSEG + NSEG
HROWS = 128
SCK = 2048
E_SCAT = E1 + E2 + N1 + N2
SCHUNKS = -(-E_SCAT // (NW * SCK))
E_SCAT_PAD = NW * SCK * SCHUNKS


def _scalar_scatter_body(idx_hbm, w_hbm, out_hbm, hist, idxb, wb, rowidx, accum):
    c = lax.axis_index("c")
    s = lax.axis_index("s")
    wid = _worker_id()

    _zero_vmem_rows(hist, HROWS)
    pltpu.sync_copy(hist.at[pl.ds(0, 8)], accum.at[pl.ds(s * 8, 8)])
    for g in range(HROWS // 16):
        rowidx[pl.ds(16 * g, 16)] = lax.iota(jnp.int32, 16) + 16 * g
    plsc.subcore_barrier()

    lanes = lax.iota(jnp.int32, 16)

    def chunk(j, _):
        base = (wid * SCHUNKS + j) * SCK
        pltpu.sync_copy(idx_hbm.at[pl.ds(base, SCK)], idxb)
        pltpu.sync_copy(w_hbm.at[pl.ds(base, SCK)], wb)

        def grp(t, _):
            iv = idxb[pl.ds(16 * t, 16)]
            wv = wb[pl.ds(16 * t, 16)]
            for l in range(16):
                i = iv[l]
                r = lax.shift_right_logical(i, 7)
                col = lax.bitwise_and(i, 127)
                colg = lax.bitwise_and(col, 112)
                vec = jnp.where(lanes == col - colg, wv[l], 0.0)
                plsc.addupdate(hist.at[r, pl.ds(colg, 16)], vec)
            return _

        lax.fori_loop(0, SCK // 16, grp, None)
        return _

    lax.fori_loop(0, SCHUNKS, chunk, None)
    pltpu.sync_copy(hist, accum.at[rowidx], add=True)
    plsc.subcore_barrier()
    pltpu.sync_copy(accum.at[pl.ds(s * 8, 8)], out_hbm.at[c, pl.ds(s * 8, 8)])


@functools.lru_cache(maxsize=None)
def _get_scalar_scatter():
    return pl.kernel(
        _scalar_scatter_body,
        out_type=jax.ShapeDtypeStruct((NC, HROWS, 128), jnp.float32),
        mesh=_get_mesh(),
        scratch_types=[
            pltpu.VMEM((HROWS, 128), jnp.float32),
            pltpu.VMEM((SCK,), jnp.int32),
            pltpu.VMEM((SCK,), jnp.float32),
            pltpu.VMEM((HROWS,), jnp.int32),
            pltpu.VMEM_SHARED((HROWS, 128), jnp.float32),
        ],
    )


@functools.lru_cache(maxsize=None)
def _make_rowagg(nrows, nchunks, gather):
    rps = nrows // NS

    def body(*refs):
        if gather:
            y_hbm, sidx_hbm, didx_hbm, w_hbm, out_hbm = refs[:5]
            sidx, didx, wb, rows, sem, accum = refs[5:]
        else:
            y_hbm, didx_hbm, out_hbm = refs[:3]
            didx, rows, sem, accum = refs[3:]
        c = lax.axis_index("c")
        s = lax.axis_index("s")
        wid = _worker_id()

        _zero_vmem_rows(rows, CK)
        _zero_shared_slice(accum, rows, CK, s * rps, rps)
        plsc.subcore_barrier()

        def chunk(j, _):
            base = (wid * nchunks + j) * CK
            pltpu.sync_copy(didx_hbm.at[pl.ds(base, CK)], didx)
            if gather:
                pltpu.sync_copy(sidx_hbm.at[pl.ds(base, CK)], sidx)
                pltpu.sync_copy(w_hbm.at[pl.ds(base, CK)], wb)
                pltpu.async_copy(y_hbm.at[sidx], rows, sem).wait()

                def scale(t, _):
                    wv = wb[pl.ds(16 * t, 16)]
                    for l in range(16):
                        wk = wv[l]
                        rr = 16 * t + l
                        for g in range(8):
                            rows[rr, pl.ds(16 * g, 16)] = (
                                rows[rr, pl.ds(16 * g, 16)] * wk)
                    return _

                lax.fori_loop(0, CK // 16, scale, None)
            else:
                pltpu.sync_copy(y_hbm.at[pl.ds(base, CK)], rows)
            pltpu.sync_copy(rows, accum.at[didx], add=True)
            return _

        lax.fori_loop(0, nchunks, chunk, None)
        plsc.subcore_barrier()
        off = 0
        while off < rps:
            sz = min(CK, rps - off)
            pltpu.sync_copy(accum.at[pl.ds(s * rps + off, sz)],
                            out_hbm.at[c, pl.ds(s * rps + off, sz)])
            off += sz

    scratch = []
    if gather:
        scratch.append(pltpu.VMEM((CK,), jnp.int32))
    scratch.append(pltpu.VMEM((CK,), jnp.int32))
    if gather:
        scratch.append(pltpu.VMEM((CK,), jnp.float32))
    scratch += [
        pltpu.VMEM((CK, 128), jnp.float32),
        pltpu.SemaphoreType.DMA,
        pltpu.VMEM_SHARED((nrows, 128), jnp.float32),
    ]
    return pl.kernel(
        body,
        out_type=jax.ShapeDtypeStruct((NC, nrows, 128), jnp.float32),
        mesh=_get_mesh(),
        scratch_types=scratch,
    )


EPAD1 = NW * CK * (-(-E1 // (NW * CK)))
EPAD2 = NW * CK * (-(-E2 // (NW * CK)))
RPAD1 = NW * CK * (-(-N1 // (NW * CK)))
RPAD2 = NW * CK * (-(-N2 // (NW * CK)))
NP1 = 128 * (-(-N1 // 128))
NP2 = 128 * (-(-N2 // 128))
NPSEG = 128 * (-(-NSEG // 128))


def _dinv_of(degp):
    deg = degp[0] + degp[1] + 1.0
    return lax.rsqrt(deg)


def _mm1_body(x_ref, w_ref, b_ref, degp_ref, y_ref, sl_ref):
    xw = jnp.dot(x_ref[...], w_ref[...], preferred_element_type=jnp.float32)
    dinv = _dinv_of(degp_ref[...])
    y_ref[...] = xw * dinv
    sl_ref[...] = xw * (dinv * dinv) + b_ref[...]


def _make_mm1(n, br):
    grid = n // br
    return pl.pallas_call(
        _mm1_body,
        grid=(grid,),
        in_specs=[
            pl.BlockSpec((br, D), lambda i: (i, 0)),
            pl.BlockSpec((D, D), lambda i: (0, 0)),
            pl.BlockSpec((1, D), lambda i: (0, 0)),
            pl.BlockSpec((2, br, 1), lambda i: (0, i, 0)),
        ],
        out_specs=[
            pl.BlockSpec((br, D), lambda i: (i, 0)),
            pl.BlockSpec((br, D), lambda i: (i, 0)),
        ],
        out_shape=[
            jax.ShapeDtypeStruct((n, D), jnp.float32),
            jax.ShapeDtypeStruct((n, D), jnp.float32),
        ],
    )


def _mid_body(aggp_ref, sl_ref, w_ref, b_ref, degp_ref, y_ref, sl2_ref, h_ref):
    dinv = _dinv_of(degp_ref[...])
    a = aggp_ref[0] + aggp_ref[1]
    h = jnp.maximum(a * dinv + sl_ref[...], 0.0)
    h_ref[...] = h
    xw = jnp.dot(h, w_ref[...], preferred_element_type=jnp.float32)
    y_ref[...] = xw * dinv
    sl2_ref[...] = xw * (dinv * dinv) + b_ref[...]


def _make_mid(n, br):
    grid = n // br
    return pl.pallas_call(
        _mid_body,
        grid=(grid,),
        in_specs=[
            pl.BlockSpec((2, br, D), lambda i: (0, i, 0)),
            pl.BlockSpec((br, D), lambda i: (i, 0)),
            pl.BlockSpec((D, D), lambda i: (0, 0)),
            pl.BlockSpec((1, D), lambda i: (0, 0)),
            pl.BlockSpec((2, br, 1), lambda i: (0, i, 0)),
        ],
        out_specs=[
            pl.BlockSpec((br, D), lambda i: (i, 0)),
            pl.BlockSpec((br, D), lambda i: (i, 0)),
            pl.BlockSpec((br, D), lambda i: (i, 0)),
        ],
        out_shape=[
            jax.ShapeDtypeStruct((n, D), jnp.float32),
            jax.ShapeDtypeStruct((n, D), jnp.float32),
            jax.ShapeDtypeStruct((n, D), jnp.float32),
        ],
    )


def _post_body(aggp_ref, sl_ref, degp_ref, h_ref):
    dinv = _dinv_of(degp_ref[...])
    h_ref[...] = jnp.maximum((aggp_ref[0] + aggp_ref[1]) * dinv + sl_ref[...], 0.0)


def _make_post(n, br):
    grid = n // br
    return pl.pallas_call(
        _post_body,
        grid=(grid,),
        in_specs=[
            pl.BlockSpec((2, br, D), lambda i: (0, i, 0)),
            pl.BlockSpec((br, D), lambda i: (i, 0)),
            pl.BlockSpec((2, br, 1), lambda i: (0, i, 0)),
        ],
        out_specs=pl.BlockSpec((br, D), lambda i: (i, 0)),
        out_shape=jax.ShapeDtypeStruct((n, D), jnp.float32),
    )


def _means_body(s1_ref, c1_ref, s2_ref, c2_ref, xp_ref, x2p_ref, comb_ref):
    c1 = jnp.maximum(c1_ref[0] + c1_ref[1], 1.0)
    c2 = jnp.maximum(c2_ref[0] + c2_ref[1], 1.0)
    xp = (s1_ref[0] + s1_ref[1]) / c1
    x2p = (s2_ref[0] + s2_ref[1]) / c2
    xp_ref[...] = xp
    x2p_ref[...] = x2p
    comb_ref[...] = xp + x2p


_means = pl.pallas_call(
    _means_body,
    in_specs=[
        pl.BlockSpec((2, NSEG, D), lambda: (0, 0, 0)),
        pl.BlockSpec((2, NSEG, 1), lambda: (0, 0, 0)),
        pl.BlockSpec((2, NSEG, D), lambda: (0, 0, 0)),
        pl.BlockSpec((2, NSEG, 1), lambda: (0, 0, 0)),
    ],
    out_specs=[
        pl.BlockSpec((NSEG, D), lambda: (0, 0)),
        pl.BlockSpec((NSEG, D), lambda: (0, 0)),
        pl.BlockSpec((NSEG, D), lambda: (0, 0)),
    ],
    out_shape=[
        jax.ShapeDtypeStruct((NSEG, D), jnp.float32),
        jax.ShapeDtypeStruct((NSEG, D), jnp.float32),
        jax.ShapeDtypeStruct((NSEG, D), jnp.float32),
    ],
)


def _ln(x, g, t):
    m = jnp.mean(x, axis=-1, keepdims=True)
    v = jnp.mean((x - m) ** 2, axis=-1, keepdims=True)
    return (x - m) * lax.rsqrt(v + 1e-5) * g + t


def _attn_body(x_ref, wq, wk, wv, wo, bq, bk, bv, bo, g1, t1, g2, t2,
               wf1, bf1, wf2, bf2, t_ref, aw_ref):
    x = x_ref[0]
    ct = (((1,), (1,)), ((), ()))
    q = lax.dot_general(x, wq[...], ct, preferred_element_type=jnp.float32) + bq[...]
    k = lax.dot_general(x, wk[...], ct, preferred_element_type=jnp.float32) + bk[...]
    v = lax.dot_general(x, wv[...], ct, preferred_element_type=jnp.float32) + bv[...]
    dh = D // H
    scale = 1.0 / jnp.sqrt(jnp.float32(dh))
    o_parts = []
    for h in range(H):
        qh = q[:, h * dh:(h + 1) * dh]
        kh = k[:, h * dh:(h + 1) * dh]
        vh = v[:, h * dh:(h + 1) * dh]
        logits = lax.dot_general(qh, kh, ct, preferred_element_type=jnp.float32) * scale
        m = jnp.max(logits, axis=-1, keepdims=True)
        e = jnp.exp(logits - m)
        aw = e / jnp.sum(e, axis=-1, keepdims=True)
        aw_ref[0, h] = aw
        o_parts.append(jnp.dot(aw, vh, preferred_element_type=jnp.float32))
    o = jnp.concatenate(o_parts, axis=-1)
    o = lax.dot_general(o, wo[...], ct, preferred_element_type=jnp.float32) + bo[...]
    hh = _ln(x + o, g1[...], t1[...])
    ff = jnp.maximum(
        lax.dot_general(hh, wf1[...], ct, preferred_element_type=jnp.float32) + bf1[...],
        0.0)
    ff = lax.dot_general(ff, wf2[...], ct, preferred_element_type=jnp.float32) + bf2[...]
    t_ref[0] = _ln(hh + ff, g2[...], t2[...])


def _make_attn():
    wspec = pl.BlockSpec((D, D), lambda i: (0, 0))
    bspec = pl.BlockSpec((1, D), lambda i: (0, 0))
    return pl.pallas_call(
        _attn_body,
        grid=(B,),
        in_specs=[pl.BlockSpec((1, NR, D), lambda i: (i, 0, 0))]
        + [wspec] * 4 + [bspec] * 4 + [bspec] * 4 + [wspec, bspec, wspec, bspec],
        out_specs=[
            pl.BlockSpec((1, NR, D), lambda i: (i, 0, 0)),
            pl.BlockSpec((1, H, NR, NR), lambda i: (i, 0, 0, 0)),
        ],
        out_shape=[
            jax.ShapeDtypeStruct((B, NR, D), jnp.float32),
            jax.ShapeDtypeStruct((B, H, NR, NR), jnp.float32),
        ],
    )


_attn = _make_attn()

KBLK = 512
KSTEPS = (NR * D) // KBLK


def _mlp_body(x_ref, w1_ref, b1_ref, bng_ref, bnb_ref, w2_ref, b2_ref,
              z_ref, out_ref):
    kk = pl.program_id(0)

    @pl.when(kk == 0)
    def _():
        z_ref[...] = jnp.broadcast_to(b1_ref[...], (B, HID))

    z_ref[...] += jnp.dot(x_ref[...], w1_ref[...], preferred_element_type=jnp.float32)

    @pl.when(kk == KSTEPS - 1)
    def _():
        z = z_ref[...] * (1.0 / jnp.sqrt(1.0 + 1e-5)) * bng_ref[...] + bnb_ref[...]
        z = jnp.where(z > 0, z, 0.01 * z)
        out_ref[...] = jnp.dot(z, w2_ref[...], preferred_element_type=jnp.float32) \
            + b2_ref[...]


_mlp = pl.pallas_call(
    _mlp_body,
    grid=(KSTEPS,),
    in_specs=[
        pl.BlockSpec((B, KBLK), lambda k: (0, k)),
        pl.BlockSpec((KBLK, HID), lambda k: (k, 0)),
        pl.BlockSpec((1, HID), lambda k: (0, 0)),
        pl.BlockSpec((1, HID), lambda k: (0, 0)),
        pl.BlockSpec((1, HID), lambda k: (0, 0)),
        pl.BlockSpec((HID, 128), lambda k: (0, 0)),
        pl.BlockSpec((1, 128), lambda k: (0, 0)),
    ],
    out_specs=[
        pl.BlockSpec((B, HID), lambda k: (0, 0)),
        pl.BlockSpec((B, 128), lambda k: (0, 0)),
    ],
    out_shape=[
        jax.ShapeDtypeStruct((B, HID), jnp.float32),
        jax.ShapeDtypeStruct((B, 128), jnp.float32),
    ],
)

_mm1_1 = _make_mm1(N1, 400)
_mm1_2 = _make_mm1(N2, NSEG)
_mid_1 = _make_mid(N1, 400)
_mid_2 = _make_mid(N2, NSEG)
_post_1 = _make_post(N1, 400)
_post_2 = _make_post(N2, NSEG)


def _pad1(a, n, val=0):
    return jnp.pad(a, (0, n - a.shape[0]), constant_values=val)


def kernel(x, node_roi, edge_index, edge_attr, batch, x2, roi2, edge_index2,
           edge_attr2, batch2, params):
    p = params

    seg1 = batch * NR + node_roi
    seg2 = batch2 * NR + roi2

    scat_idx = jnp.concatenate([
        edge_index[1],
        edge_index2[1] + N1,
        seg1 + (N1 + N2),
        seg2 + (N1 + N2 + NSEG),
    ])
    scat_w = jnp.concatenate([
        edge_attr, edge_attr2,
        jnp.ones((N1,), jnp.float32), jnp.ones((N2,), jnp.float32),
    ])
    scat_idx = _pad1(scat_idx, E_SCAT_PAD)
    scat_w = _pad1(scat_w, E_SCAT_PAD)
    hist = _get_scalar_scatter()(scat_idx, scat_w)
    _agg1 = _make_rowagg(NP1, EPAD1 // (NW * CK), True)
    _agg2 = _make_rowagg(NP2, EPAD2 // (NW * CK), True)
    _pool1 = _make_rowagg(NPSEG, RPAD1 // (NW * CK), False)
    _pool2 = _make_rowagg(NPSEG, RPAD2 // (NW * CK), False)
    flat = hist.reshape(NC, HROWS * 128)
    deg1p = flat[:, :N1].reshape(NC, N1, 1)
    deg2p = flat[:, N1:N1 + N2].reshape(NC, N2, 1)
    cnt1p = flat[:, N1 + N2:N1 + N2 + NSEG].reshape(NC, NSEG, 1)
    cnt2p = flat[:, N1 + N2 + NSEG:N1 + N2 + 2 * NSEG].reshape(NC, NSEG, 1)

    s1 = _pad1(edge_index[0], EPAD1)
    d1 = _pad1(edge_index[1], EPAD1)
    w1 = _pad1(edge_attr, EPAD1)
    s2 = _pad1(edge_index2[0], EPAD2)
    d2 = _pad1(edge_index2[1], EPAD2)
    w2e = _pad1(edge_attr2, EPAD2)
    seg1p = _pad1(seg1, RPAD1)
    seg2p = _pad1(seg2, RPAD2)

    b1 = p['b1'].reshape(1, D); b2 = p['b2'].reshape(1, D)
    y, sl = _mm1_1(x, p['W1'], b1, deg1p)
    aggp = _agg1(y, s1, d1, w1)
    y2, sl2, _h1 = _mid_1(aggp[:, :N1], sl, p['W2'], b2, deg1p)
    aggp2 = _agg1(y2, s1, d1, w1)
    h1 = _post_1(aggp2[:, :N1], sl2, deg1p)
    h1p = jnp.pad(h1, ((0, RPAD1 - N1), (0, 0)))
    sum1p = _pool1(h1p, seg1p)

    b1r = p['b1r'].reshape(1, D); b2r = p['b2r'].reshape(1, D)
    yr, slr = _mm1_2(x2, p['W1r'], b1r, deg2p)
    aggpr = _agg2(yr, s2, d2, w2e)
    y2r, sl2r, _h = _mid_2(aggpr[:, :N2], slr, p['W2r'], b2r, deg2p)
    aggp2r = _agg2(y2r, s2, d2, w2e)
    h2 = _post_2(aggp2r[:, :N2], sl2r, deg2p)
    h2p = jnp.pad(h2, ((0, RPAD2 - N2), (0, 0)))
    sum2p = _pool2(h2p, seg2p)

    xp2d, x2p2d, comb2d = _means(sum1p[:, :NSEG], cnt1p, sum2p[:, :NSEG], cnt2p)
    xp = xp2d.reshape(B, NR, D)
    x2p = x2p2d.reshape(B, NR, D)
    combined = comb2d.reshape(B, NR, D)

    r = lambda v: v.reshape(1, D)
    t_out, aw = _attn(combined, p['Wq'], p['Wk'], p['Wv'], p['Wo'],
                      r(p['bq']), r(p['bk']), r(p['bv']), r(p['bo']),
                      r(p['g1']), r(p['t1']), r(p['g2']), r(p['t2']),
                      p['Wf1'], r(p['bf1']), p['Wf2'], r(p['bf2']))

    flat_t = t_out.reshape(B, NR * D)
    wc2p = jnp.pad(p['Wc2'], ((0, 0), (0, 128 - OUT)))
    bc2p = jnp.pad(p['bc2'], (0, 128 - OUT)).reshape(1, 128)
    _z, outp = _mlp(flat_t, p['Wc1'], p['bc1'].reshape(1, HID),
                    p['bng'].reshape(1, HID), p['bnb'].reshape(1, HID),
                    wc2p, bc2p)
    out = outp[:, :OUT]

    return (out, xp, x2p, combined, t_out, aw)
